# baseline (speedup 1.0000x reference)
import jax
import jax.numpy as jnp
from jax.experimental import pallas as pl
from jax.experimental.pallas import tpu as pltpu

N_Q, N_R, D, H = 4096, 16384, 1024, 128
BQ = 512
BR = 4096
CH = 128
NCH = N_R // CH
CC = 2048
NCC = N_R // CC
CPC = CC // CH

_HI = jax.lax.Precision.HIGHEST
_DF = jax.lax.Precision.DEFAULT
_BF = jnp.bfloat16
_DNT = (((1,), (1,)), ((), ()))


def _remb_body(r_ref, w_ref, b_ref, o_ref):
    acc = jax.lax.dot(r_ref[...].astype(_BF), w_ref[...].astype(_BF),
                      precision=_DF, preferred_element_type=jnp.float32)
    acc = acc + b_ref[...]
    n = jnp.sqrt(jnp.sum(acc * acc, axis=1, keepdims=True))
    o_ref[...] = (acc / jnp.maximum(n, 1e-12)).astype(_BF)


def _select_body(q_ref, t_ref, l_ref, w_ref, b_ref, re_ref, u_ref,
                 bind_ref, chid_ref, fold_ref, sel_ref, lp_ref, eb_ref):
    w = w_ref[...]
    acc = jax.lax.dot(q_ref[...].astype(_BF), w[0:D, :], precision=_DF,
                      preferred_element_type=jnp.float32)
    acc += jax.lax.dot(t_ref[...].astype(_BF), w[D:2 * D, :], precision=_DF,
                       preferred_element_type=jnp.float32)
    acc += jax.lax.dot(l_ref[...].astype(_BF), w[2 * D:3 * D, :], precision=_DF,
                       preferred_element_type=jnp.float32)
    acc = acc + b_ref[...]
    n = jnp.sqrt(jnp.sum(acc * acc, axis=1, keepdims=True))
    emb = (acc / jnp.maximum(n, 1e-12)).astype(_BF)

    bind = bind_ref[...]
    s_parts = []
    for c in range(NCC):
        lgc = jax.lax.dot_general(emb, re_ref[c * CC:(c + 1) * CC, :], _DNT,
                                  precision=_DF,
                                  preferred_element_type=jnp.float32)
        ebc = jnp.exp(lgc).astype(_BF)
        eb_ref[:, c * CC:(c + 1) * CC] = ebc
        s_parts.append(jax.lax.dot(ebc, bind, precision=_DF,
                                   preferred_element_type=jnp.float32))
    s = jnp.concatenate(s_parts, axis=1)

    tri = (jax.lax.broadcasted_iota(jnp.int32, (NCH, NCH), 0)
           <= jax.lax.broadcasted_iota(jnp.int32, (NCH, NCH), 1)
           ).astype(jnp.float32)
    cs = jax.lax.dot(s, tri, precision=_HI,
                     preferred_element_type=jnp.float32)
    z = cs[:, NCH - 1:NCH]
    thr = u_ref[...] * z
    cse = cs - s
    cross = jnp.logical_and(cse <= thr, cs > thr)
    crossf = cross.astype(jnp.float32)
    iota_c = jax.lax.broadcasted_iota(jnp.int32, (BQ, NCH), 1)
    ci = jnp.sum(jnp.where(cross, iota_c, 0), axis=1, keepdims=True)
    base = jnp.sum(crossf * cse, axis=1, keepdims=True)

    cib = ci.astype(_BF)
    fold = fold_ref[...]
    zero = jnp.zeros((), _BF)
    xsel = jnp.zeros((BQ, CH), jnp.float32)
    for c in range(NCC):
        exc = jnp.where(chid_ref[:, c * CC:(c + 1) * CC] == cib,
                        eb_ref[:, c * CC:(c + 1) * CC], zero)
        xsel += jax.lax.dot(exc, fold, precision=_DF,
                            preferred_element_type=jnp.float32)

    cx = jax.lax.dot(xsel, tri, precision=_HI,
                     preferred_element_type=jnp.float32)
    below = (base + cx <= thr)
    fine = jnp.sum(below.astype(jnp.int32), axis=1, keepdims=True)

    lane = jnp.minimum(fine, CH - 1)
    lmask = jax.lax.broadcasted_iota(jnp.int32, (BQ, CH), 1) == lane
    esel = jnp.sum(jnp.where(lmask, xsel, 0.0), axis=1, keepdims=True)

    nocross = z <= thr
    esel = jnp.where(nocross, eb_ref[:, 0:1].astype(jnp.float32), esel)
    sel = jnp.where(nocross, 0, ci * CH + fine)
    sel_ref[...] = sel
    lp_ref[...] = jnp.log(esel) - jnp.log(z)


def kernel(queries, tasks, llms, reasonings, W_qtl, b_qtl, W_r, b_r,
           random_num):
    w_rt = W_r.T
    b_r2 = b_r.reshape(1, H)
    r_emb = pl.pallas_call(
        _remb_body,
        grid=(N_R // BR,),
        in_specs=[
            pl.BlockSpec((BR, D), lambda i: (i, 0)),
            pl.BlockSpec((D, H), lambda i: (0, 0)),
            pl.BlockSpec((1, H), lambda i: (0, 0)),
        ],
        out_specs=pl.BlockSpec((BR, H), lambda i: (i, 0)),
        out_shape=jax.ShapeDtypeStruct((N_R, H), _BF),
    )(reasonings, w_rt, b_r2)

    w_qt = W_qtl.T.astype(_BF)
    b_q2 = b_qtl.reshape(1, H)

    jj = jnp.arange(CC, dtype=jnp.int32)
    bind = (jj[:, None] // CH == jnp.arange(CPC, dtype=jnp.int32)[None, :]
            ).astype(_BF)
    fold = (jj[:, None] % CH == jnp.arange(CH, dtype=jnp.int32)[None, :]
            ).astype(_BF)
    chid = (jnp.arange(N_R, dtype=jnp.int32)[None, :] // CH).astype(_BF)

    sel, lp = pl.pallas_call(
        _select_body,
        grid=(N_Q // BQ,),
        in_specs=[
            pl.BlockSpec((BQ, D), lambda i: (i, 0)),
            pl.BlockSpec((BQ, D), lambda i: (i, 0)),
            pl.BlockSpec((BQ, D), lambda i: (i, 0)),
            pl.BlockSpec((3 * D, H), lambda i: (0, 0)),
            pl.BlockSpec((1, H), lambda i: (0, 0)),
            pl.BlockSpec((N_R, H), lambda i: (0, 0)),
            pl.BlockSpec((BQ, 1), lambda i: (i, 0)),
            pl.BlockSpec((CC, CPC), lambda i: (0, 0)),
            pl.BlockSpec((1, N_R), lambda i: (0, 0)),
            pl.BlockSpec((CC, CH), lambda i: (0, 0)),
        ],
        out_specs=[
            pl.BlockSpec((BQ, 1), lambda i: (i, 0)),
            pl.BlockSpec((BQ, 1), lambda i: (i, 0)),
        ],
        out_shape=[
            jax.ShapeDtypeStruct((N_Q, 1), jnp.int32),
            jax.ShapeDtypeStruct((N_Q, 1), jnp.float32),
        ],
        scratch_shapes=[pltpu.VMEM((BQ, N_R), _BF)],
    )(queries, tasks, llms, w_qt, b_q2, r_emb, random_num, bind, chid, fold)
    return (sel[:, 0], lp)


# speedup vs baseline: 1.2669x; 1.2669x over previous
import jax
import jax.numpy as jnp
from jax.experimental import pallas as pl
from jax.experimental.pallas import tpu as pltpu

N_Q, N_R, D, H = 4096, 16384, 1024, 128
BQ = 256
BR = 4096
CH = 128
NCH = N_R // CH
CC = 4096
NCC = N_R // CC
CPC = CC // CH

_HI = jax.lax.Precision.HIGHEST
_DF = jax.lax.Precision.DEFAULT
_BF = jnp.bfloat16
_DNT = (((1,), (1,)), ((), ()))


def _remb_body(r_ref, w_ref, b_ref, o_ref):
    acc = jax.lax.dot(r_ref[...].astype(_BF), w_ref[...].astype(_BF),
                      precision=_DF, preferred_element_type=jnp.float32)
    acc = acc + b_ref[...]
    n = jnp.sqrt(jnp.sum(acc * acc, axis=1, keepdims=True))
    o_ref[...] = (acc / jnp.maximum(n, 1e-12)).astype(_BF)


def _select_body(q_ref, t_ref, l_ref, w_ref, b_ref, re_ref, u_ref,
                 bind_ref, chid_ref, fold_ref, sel_ref, lp_ref, eb_ref):
    w = w_ref[...]
    acc = jax.lax.dot(q_ref[...].astype(_BF), w[0:D, :], precision=_DF,
                      preferred_element_type=jnp.float32)
    acc += jax.lax.dot(t_ref[...].astype(_BF), w[D:2 * D, :], precision=_DF,
                       preferred_element_type=jnp.float32)
    acc += jax.lax.dot(l_ref[...].astype(_BF), w[2 * D:3 * D, :], precision=_DF,
                       preferred_element_type=jnp.float32)
    acc = acc + b_ref[...]
    n = jnp.sqrt(jnp.sum(acc * acc, axis=1, keepdims=True))
    emb = (acc / jnp.maximum(n, 1e-12)).astype(_BF)

    bind = bind_ref[...]
    s_parts = []
    for c in range(NCC):
        lgc = jax.lax.dot_general(emb, re_ref[c * CC:(c + 1) * CC, :], _DNT,
                                  precision=_DF,
                                  preferred_element_type=jnp.float32)
        ebc = jnp.exp(lgc).astype(_BF)
        eb_ref[:, c * CC:(c + 1) * CC] = ebc
        s_parts.append(jax.lax.dot(ebc, bind, precision=_DF,
                                   preferred_element_type=jnp.float32))
    s = jnp.concatenate(s_parts, axis=1)

    tri = (jax.lax.broadcasted_iota(jnp.int32, (NCH, NCH), 0)
           <= jax.lax.broadcasted_iota(jnp.int32, (NCH, NCH), 1)
           ).astype(jnp.float32)
    cs = jax.lax.dot(s, tri, precision=_HI,
                     preferred_element_type=jnp.float32)
    z = cs[:, NCH - 1:NCH]
    thr = u_ref[...] * z
    cse = cs - s
    cross = jnp.logical_and(cse <= thr, cs > thr)
    crossf = cross.astype(jnp.float32)
    iota_c = jax.lax.broadcasted_iota(jnp.int32, (BQ, NCH), 1)
    ci = jnp.sum(jnp.where(cross, iota_c, 0), axis=1, keepdims=True)
    base = jnp.sum(crossf * cse, axis=1, keepdims=True)

    cib = ci.astype(_BF)
    fold = fold_ref[...]
    zero = jnp.zeros((), _BF)
    xsel = jnp.zeros((BQ, CH), jnp.float32)
    for c in range(NCC):
        exc = jnp.where(chid_ref[:, c * CC:(c + 1) * CC] == cib,
                        eb_ref[:, c * CC:(c + 1) * CC], zero)
        xsel += jax.lax.dot(exc, fold, precision=_DF,
                            preferred_element_type=jnp.float32)

    cx = jax.lax.dot(xsel, tri, precision=_HI,
                     preferred_element_type=jnp.float32)
    below = (base + cx <= thr)
    fine = jnp.sum(below.astype(jnp.int32), axis=1, keepdims=True)

    lane = jnp.minimum(fine, CH - 1)
    lmask = jax.lax.broadcasted_iota(jnp.int32, (BQ, CH), 1) == lane
    esel = jnp.sum(jnp.where(lmask, xsel, 0.0), axis=1, keepdims=True)

    nocross = z <= thr
    esel = jnp.where(nocross, eb_ref[:, 0:1].astype(jnp.float32), esel)
    sel = jnp.where(nocross, 0, ci * CH + fine)
    sel_ref[...] = sel
    lp_ref[...] = jnp.log(esel) - jnp.log(z)


def kernel(queries, tasks, llms, reasonings, W_qtl, b_qtl, W_r, b_r,
           random_num):
    w_rt = W_r.T
    b_r2 = b_r.reshape(1, H)
    r_emb = pl.pallas_call(
        _remb_body,
        grid=(N_R // BR,),
        in_specs=[
            pl.BlockSpec((BR, D), lambda i: (i, 0)),
            pl.BlockSpec((D, H), lambda i: (0, 0)),
            pl.BlockSpec((1, H), lambda i: (0, 0)),
        ],
        out_specs=pl.BlockSpec((BR, H), lambda i: (i, 0)),
        out_shape=jax.ShapeDtypeStruct((N_R, H), _BF),
    )(reasonings, w_rt, b_r2)

    w_qt = W_qtl.T.astype(_BF)
    b_q2 = b_qtl.reshape(1, H)

    jj = jnp.arange(CC, dtype=jnp.int32)
    bind = (jj[:, None] // CH == jnp.arange(CPC, dtype=jnp.int32)[None, :]
            ).astype(_BF)
    fold = (jj[:, None] % CH == jnp.arange(CH, dtype=jnp.int32)[None, :]
            ).astype(_BF)
    chid = (jnp.arange(N_R, dtype=jnp.int32)[None, :] // CH).astype(_BF)

    sel, lp = pl.pallas_call(
        _select_body,
        grid=(N_Q // BQ,),
        in_specs=[
            pl.BlockSpec((BQ, D), lambda i: (i, 0)),
            pl.BlockSpec((BQ, D), lambda i: (i, 0)),
            pl.BlockSpec((BQ, D), lambda i: (i, 0)),
            pl.BlockSpec((3 * D, H), lambda i: (0, 0)),
            pl.BlockSpec((1, H), lambda i: (0, 0)),
            pl.BlockSpec((N_R, H), lambda i: (0, 0)),
            pl.BlockSpec((BQ, 1), lambda i: (i, 0)),
            pl.BlockSpec((CC, CPC), lambda i: (0, 0)),
            pl.BlockSpec((1, N_R), lambda i: (0, 0)),
            pl.BlockSpec((CC, CH), lambda i: (0, 0)),
        ],
        out_specs=[
            pl.BlockSpec((BQ, 1), lambda i: (i, 0)),
            pl.BlockSpec((BQ, 1), lambda i: (i, 0)),
        ],
        out_shape=[
            jax.ShapeDtypeStruct((N_Q, 1), jnp.int32),
            jax.ShapeDtypeStruct((N_Q, 1), jnp.float32),
        ],
        scratch_shapes=[pltpu.VMEM((BQ, N_R), _BF)],
    )(queries, tasks, llms, w_qt, b_q2, r_emb, random_num, bind, chid, fold)
    return (sel[:, 0], lp)
